# Initial kernel scaffold; baseline (speedup 1.0000x reference)
import functools

import jax
import jax.numpy as jnp
from jax import lax
from jax.experimental import pallas as pl
from jax.experimental.pallas import tpu as pltpu
from jax.experimental.pallas import tpu_sc as plsc

_NC = 2
_NS = 16
_NW = _NC * _NS
_BN = 512
_CH = 80


def _table_body(h_ref, w_ref, g_ref, o_ref):
    hb = h_ref[...]
    w = w_ref[0]
    g = g_ref[...]
    t = jnp.dot(hb, w, preferred_element_type=jnp.float32)
    logit = jnp.sum(hb * g, axis=1, keepdims=True)
    o_ref[0] = t * jax.nn.sigmoid(logit)


def _make_table(hpad, W, gate2d, npad, d, r):
    return pl.pallas_call(
        _table_body,
        grid=(r, npad // _BN),
        in_specs=[
            pl.BlockSpec((_BN, d), lambda ri, nb: (nb, 0)),
            pl.BlockSpec((1, d, d), lambda ri, nb: (ri, 0, 0)),
            pl.BlockSpec((1, d), lambda ri, nb: (ri, 0)),
        ],
        out_specs=pl.BlockSpec((1, _BN, d), lambda ri, nb: (ri, nb, 0)),
        out_shape=jax.ShapeDtypeStruct((r, npad, d), jnp.float32),
    )(hpad, W, gate2d)


def _combine_body(p_ref, h_ref, loop_ref, b_ref, o_ref, *, act):
    acc = p_ref[0] + p_ref[1] + b_ref[...] + jnp.dot(
        h_ref[...], loop_ref[...], preferred_element_type=jnp.float32)
    if act:
        acc = jnp.maximum(acc, 0.0)
    o_ref[...] = acc


def _combine(partials, hpad, loopW, b2d, act, npad, d):
    return pl.pallas_call(
        functools.partial(_combine_body, act=act),
        grid=(npad // _BN,),
        in_specs=[
            pl.BlockSpec((2, _BN, d), lambda nb: (0, nb, 0)),
            pl.BlockSpec((_BN, d), lambda nb: (nb, 0)),
            pl.BlockSpec((d, d), lambda nb: (0, 0)),
            pl.BlockSpec((1, d), lambda nb: (0, 0)),
        ],
        out_specs=pl.BlockSpec((_BN, d), lambda nb: (nb, 0)),
        out_shape=jax.ShapeDtypeStruct((npad, d), jnp.float32),
    )(partials, hpad, loopW, b2d)


def _edge_agg(src, rel, dst, norm, table_flat, npad, d):
    epad = src.shape[0]
    epw = epad // _NW
    nch = epw // _CH
    rpt = npad // _NS
    mesh = plsc.VectorSubcoreMesh(
        core_axis_name="c", subcore_axis_name="s", num_cores=_NC)

    @functools.partial(
        pl.kernel, mesh=mesh,
        out_type=jax.ShapeDtypeStruct((_NC, npad, d), jnp.float32),
        scratch_types=[
            pltpu.VMEM((_CH,), jnp.int32),
            pltpu.VMEM((_CH,), jnp.int32),
            pltpu.VMEM((_CH,), jnp.int32),
            pltpu.VMEM((_CH,), jnp.float32),
            pltpu.VMEM((_CH,), jnp.int32),
            pltpu.VMEM((_CH, d), jnp.float32),
            pltpu.VMEM((16, d), jnp.float32),
            pltpu.VMEM_SHARED((npad, d), jnp.float32),
            pltpu.SemaphoreType.DMA,
        ],
    )
    def body(src_h, rel_h, dst_h, norm_h, tab_h, out_h,
             src_v, rel_v, dst_v, norm_v, gidx_v, rows_v, zero_v, agg_s, sem):
        cid = lax.axis_index("c")
        sid = lax.axis_index("s")
        wid = sid * _NC + cid

        zvec = jnp.zeros((16,), jnp.float32)
        for i in range(16):
            for j in range(d // 16):
                zero_v[i, pl.ds(j * 16, 16)] = zvec

        def zloop(i, c):
            pltpu.sync_copy(zero_v, agg_s.at[pl.ds(sid * rpt + i * 16, 16)])
            return c
        lax.fori_loop(0, rpt // 16, zloop, 0)
        plsc.subcore_barrier()

        wbase = wid * epw

        def chunk(ci, c):
            base = wbase + ci * _CH
            pltpu.sync_copy(src_h.at[pl.ds(base, _CH)], src_v)
            pltpu.sync_copy(rel_h.at[pl.ds(base, _CH)], rel_v)
            pltpu.sync_copy(dst_h.at[pl.ds(base, _CH)], dst_v)
            pltpu.sync_copy(norm_h.at[pl.ds(base, _CH)], norm_v)
            for k in range(_CH // 16):
                s = pl.ds(k * 16, 16)
                gidx_v[s] = rel_v[s] * npad + src_v[s]
            pltpu.async_copy(tab_h.at[gidx_v], rows_v, sem).wait()

            def scale(e, c2):
                nv = plsc.load_gather(norm_v, [jnp.full((16,), e, jnp.int32)])
                for j in range(d // 16):
                    sl = pl.ds(j * 16, 16)
                    rows_v[e, sl] = rows_v[e, sl] * nv
                return c2
            lax.fori_loop(0, _CH, scale, 0)
            pltpu.sync_copy(rows_v, agg_s.at[dst_v], add=True)
            return c
        lax.fori_loop(0, nch, chunk, 0)
        plsc.subcore_barrier()

        pltpu.sync_copy(agg_s.at[pl.ds(sid * rpt, rpt)],
                        out_h.at[cid, pl.ds(sid * rpt, rpt)])

    return body(src, rel, dst, norm, table_flat)


def kernel(x, edge_index, rel_type, edge_norm, W0, b0, loop0, gate0,
           W1, b1, loop1, gate1):
    n, d = x.shape
    r = W0.shape[0]
    e = rel_type.shape[0]
    npad = -(-n // _BN) * _BN
    epad = -(-e // (_NW * _CH)) * (_NW * _CH)

    src = edge_index[0]
    dst = edge_index[1]
    rel = rel_type
    norm = edge_norm[:, 0]
    if epad != e:
        pad = epad - e
        src = jnp.pad(src, (0, pad))
        dst = jnp.pad(dst, (0, pad))
        rel = jnp.pad(rel, (0, pad))
        norm = jnp.pad(norm, (0, pad))
    hpad = jnp.pad(x, ((0, npad - n), (0, 0)))

    t0 = _make_table(hpad, W0, gate0[:, :, 0], npad, d, r)
    p0 = _edge_agg(src, rel, dst, norm, t0.reshape(r * npad, d), npad, d)
    h1 = _combine(p0, hpad, loop0, b0.reshape(1, d), True, npad, d)

    t1 = _make_table(h1, W1, gate1[:, :, 0], npad, d, r)
    p1 = _edge_agg(src, rel, dst, norm, t1.reshape(r * npad, d), npad, d)
    out = _combine(p1, h1, loop1, b1.reshape(1, d), False, npad, d)
    return out[:n]


# speedup vs baseline: 7.4785x; 7.4785x over previous
import functools

import jax
import jax.numpy as jnp
from jax import lax
from jax.experimental import pallas as pl
from jax.experimental.pallas import tpu as pltpu
from jax.experimental.pallas import tpu_sc as plsc

_NC = 2
_NS = 16
_NW = _NC * _NS
_BN = 512
_CH = 80


def _table_body(h_ref, w_ref, g_ref, o_ref):
    hb = h_ref[...]
    w = w_ref[0]
    g = g_ref[0]
    t = jnp.dot(hb, w, preferred_element_type=jnp.float32)
    logit = jnp.sum(hb * g, axis=1, keepdims=True)
    o_ref[0] = t * jax.nn.sigmoid(logit)


def _make_table(hpad, W, gate3d, npad, d, r):
    return pl.pallas_call(
        _table_body,
        grid=(r, npad // _BN),
        in_specs=[
            pl.BlockSpec((_BN, d), lambda ri, nb: (nb, 0)),
            pl.BlockSpec((1, d, d), lambda ri, nb: (ri, 0, 0)),
            pl.BlockSpec((1, 1, d), lambda ri, nb: (ri, 0, 0)),
        ],
        out_specs=pl.BlockSpec((1, _BN, d), lambda ri, nb: (ri, nb, 0)),
        out_shape=jax.ShapeDtypeStruct((r, npad, d), jnp.float32),
    )(hpad, W, gate3d)


def _bcast16_body(n_ref, o_ref):
    o_ref[...] = jnp.broadcast_to(n_ref[...], o_ref.shape)


def _bcast16(norm, epad):
    eb = _NW * _CH
    return pl.pallas_call(
        _bcast16_body,
        grid=(epad // eb,),
        in_specs=[pl.BlockSpec((eb, 1), lambda i: (i, 0))],
        out_specs=pl.BlockSpec((eb, 16), lambda i: (i, 0)),
        out_shape=jax.ShapeDtypeStruct((epad, 16), jnp.float32),
    )(norm.reshape(epad, 1))


def _combine_body(p_ref, h_ref, loop_ref, b_ref, o_ref, *, act):
    acc = p_ref[0] + p_ref[1] + b_ref[...] + jnp.dot(
        h_ref[...], loop_ref[...], preferred_element_type=jnp.float32)
    if act:
        acc = jnp.maximum(acc, 0.0)
    o_ref[...] = acc


def _combine(partials, hpad, loopW, b2d, act, npad, d):
    return pl.pallas_call(
        functools.partial(_combine_body, act=act),
        grid=(npad // _BN,),
        in_specs=[
            pl.BlockSpec((2, _BN, d), lambda nb: (0, nb, 0)),
            pl.BlockSpec((_BN, d), lambda nb: (nb, 0)),
            pl.BlockSpec((d, d), lambda nb: (0, 0)),
            pl.BlockSpec((1, d), lambda nb: (0, 0)),
        ],
        out_specs=pl.BlockSpec((_BN, d), lambda nb: (nb, 0)),
        out_shape=jax.ShapeDtypeStruct((npad, d), jnp.float32),
    )(partials, hpad, loopW, b2d)


def _edge_agg(src, rel, dst, normx, table_flat, npad, d):
    epad = src.shape[0]
    epw = epad // _NW
    nch = epw // _CH
    rpt = npad // _NS
    mesh = plsc.VectorSubcoreMesh(
        core_axis_name="c", subcore_axis_name="s", num_cores=_NC)

    @functools.partial(
        pl.kernel, mesh=mesh,
        out_type=jax.ShapeDtypeStruct((_NC, npad, d), jnp.float32),
        scratch_types=[
            pltpu.VMEM((_CH,), jnp.int32),
            pltpu.VMEM((_CH,), jnp.int32),
            pltpu.VMEM((_CH,), jnp.int32),
            pltpu.VMEM((_CH, 16), jnp.float32),
            pltpu.VMEM((_CH,), jnp.int32),
            pltpu.VMEM((_CH, d), jnp.float32),
            pltpu.VMEM((16, d), jnp.float32),
            pltpu.VMEM_SHARED((npad, d), jnp.float32),
            pltpu.SemaphoreType.DMA,
        ],
    )
    def body(src_h, rel_h, dst_h, normx_h, tab_h, out_h,
             src_v, rel_v, dst_v, normx_v, gidx_v, rows_v, zero_v,
             agg_s, sem):
        cid = lax.axis_index("c")
        sid = lax.axis_index("s")
        wid = sid * _NC + cid

        zvec = jnp.zeros((16,), jnp.float32)
        for i in range(16):
            for j in range(d // 16):
                zero_v[i, pl.ds(j * 16, 16)] = zvec

        def zloop(i, c):
            pltpu.sync_copy(zero_v, agg_s.at[pl.ds(sid * rpt + i * 16, 16)])
            return c
        lax.fori_loop(0, rpt // 16, zloop, 0)
        plsc.subcore_barrier()

        wbase = wid * epw

        def chunk(ci, c):
            base = wbase + ci * _CH
            pltpu.sync_copy(src_h.at[pl.ds(base, _CH)], src_v)
            pltpu.sync_copy(rel_h.at[pl.ds(base, _CH)], rel_v)
            pltpu.sync_copy(dst_h.at[pl.ds(base, _CH)], dst_v)
            pltpu.sync_copy(normx_h.at[pl.ds(base, _CH)], normx_v)
            for k in range(_CH // 16):
                s = pl.ds(k * 16, 16)
                gidx_v[s] = rel_v[s] * npad + src_v[s]
            pltpu.async_copy(tab_h.at[gidx_v], rows_v, sem).wait()

            def scale(e, c2):
                nv = normx_v[e]
                for j in range(d // 16):
                    sl = pl.ds(j * 16, 16)
                    rows_v[e, sl] = rows_v[e, sl] * nv
                return c2
            lax.fori_loop(0, _CH, scale, 0)
            pltpu.sync_copy(rows_v, agg_s.at[dst_v], add=True)
            return c
        lax.fori_loop(0, nch, chunk, 0)
        plsc.subcore_barrier()

        pltpu.sync_copy(agg_s.at[pl.ds(sid * rpt, rpt)],
                        out_h.at[cid, pl.ds(sid * rpt, rpt)])

    return body(src, rel, dst, normx, table_flat)


def kernel(x, edge_index, rel_type, edge_norm, W0, b0, loop0, gate0,
           W1, b1, loop1, gate1):
    n, d = x.shape
    r = W0.shape[0]
    e = rel_type.shape[0]
    npad = -(-n // _BN) * _BN
    epad = -(-e // (_NW * _CH)) * (_NW * _CH)

    src = edge_index[0]
    dst = edge_index[1]
    rel = rel_type
    norm = edge_norm[:, 0]
    if epad != e:
        pad = epad - e
        src = jnp.pad(src, (0, pad))
        dst = jnp.pad(dst, (0, pad))
        rel = jnp.pad(rel, (0, pad))
        norm = jnp.pad(norm, (0, pad))
    hpad = jnp.pad(x, ((0, npad - n), (0, 0)))
    normx = _bcast16(norm, epad)

    t0 = _make_table(hpad, W0, gate0.reshape(r, 1, d), npad, d, r)
    p0 = _edge_agg(src, rel, dst, normx, t0.reshape(r * npad, d), npad, d)
    h1 = _combine(p0, hpad, loop0, b0.reshape(1, d), True, npad, d)

    t1 = _make_table(h1, W1, gate1.reshape(r, 1, d), npad, d, r)
    p1 = _edge_agg(src, rel, dst, normx, t1.reshape(r * npad, d), npad, d)
    out = _combine(p1, h1, loop1, b1.reshape(1, d), False, npad, d)
    return out[:n]


# speedup vs baseline: 8.5139x; 1.1384x over previous
import functools

import jax
import jax.numpy as jnp
from jax import lax
from jax.experimental import pallas as pl
from jax.experimental.pallas import tpu as pltpu
from jax.experimental.pallas import tpu_sc as plsc

_NC = 2
_NS = 16
_NW = _NC * _NS
_BN = 512
_CH = 48


def _table_body(h_ref, w_ref, g_ref, o_ref):
    hb = h_ref[...]
    w = w_ref[0]
    g = g_ref[0]
    t = jnp.dot(hb, w, preferred_element_type=jnp.float32)
    logit = jnp.sum(hb * g, axis=1, keepdims=True)
    o_ref[0] = t * jax.nn.sigmoid(logit)


def _make_table(hpad, W, gate3d, npad, d, r):
    return pl.pallas_call(
        _table_body,
        grid=(r, npad // _BN),
        in_specs=[
            pl.BlockSpec((_BN, d), lambda ri, nb: (nb, 0)),
            pl.BlockSpec((1, d, d), lambda ri, nb: (ri, 0, 0)),
            pl.BlockSpec((1, 1, d), lambda ri, nb: (ri, 0, 0)),
        ],
        out_specs=pl.BlockSpec((1, _BN, d), lambda ri, nb: (ri, nb, 0)),
        out_shape=jax.ShapeDtypeStruct((r, npad, d), jnp.float32),
    )(hpad, W, gate3d)


def _edge_prep_body(n_ref, src_ref, rel_ref, nx_ref, gidx_ref, *, npad):
    nx_ref[...] = jnp.broadcast_to(n_ref[...], nx_ref.shape)
    gidx_ref[...] = rel_ref[...] * npad + src_ref[...]


def _edge_prep(norm, src, rel, epad, npad):
    eb = 1024
    return pl.pallas_call(
        functools.partial(_edge_prep_body, npad=npad),
        grid=(epad // eb,),
        in_specs=[
            pl.BlockSpec((eb, 1), lambda i: (i, 0)),
            pl.BlockSpec((eb // 128, 128), lambda i: (i, 0)),
            pl.BlockSpec((eb // 128, 128), lambda i: (i, 0)),
        ],
        out_specs=[
            pl.BlockSpec((eb, 16), lambda i: (i, 0)),
            pl.BlockSpec((eb // 128, 128), lambda i: (i, 0)),
        ],
        out_shape=[
            jax.ShapeDtypeStruct((epad, 16), jnp.float32),
            jax.ShapeDtypeStruct((epad // 128, 128), jnp.int32),
        ],
    )(norm.reshape(epad, 1), src.reshape(epad // 128, 128),
      rel.reshape(epad // 128, 128))


def _combine_body(p_ref, h_ref, loop_ref, b_ref, o_ref, *, act):
    acc = p_ref[0] + p_ref[1] + b_ref[...] + jnp.dot(
        h_ref[...], loop_ref[...], preferred_element_type=jnp.float32)
    if act:
        acc = jnp.maximum(acc, 0.0)
    o_ref[...] = acc


def _combine(partials, hpad, loopW, b2d, act, npad, d):
    return pl.pallas_call(
        functools.partial(_combine_body, act=act),
        grid=(npad // _BN,),
        in_specs=[
            pl.BlockSpec((2, _BN, d), lambda nb: (0, nb, 0)),
            pl.BlockSpec((_BN, d), lambda nb: (nb, 0)),
            pl.BlockSpec((d, d), lambda nb: (0, 0)),
            pl.BlockSpec((1, d), lambda nb: (0, 0)),
        ],
        out_specs=pl.BlockSpec((_BN, d), lambda nb: (nb, 0)),
        out_shape=jax.ShapeDtypeStruct((npad, d), jnp.float32),
    )(partials, hpad, loopW, b2d)


def _edge_agg(gidx3, dst3, normx, table_flat, npad, d):
    nw, nch, _ = gidx3.shape
    epw = nch * _CH
    rpt = npad // _NS
    mesh = plsc.VectorSubcoreMesh(
        core_axis_name="c", subcore_axis_name="s", num_cores=_NC)

    @functools.partial(
        pl.kernel, mesh=mesh,
        out_type=jax.ShapeDtypeStruct((_NC, npad, d), jnp.float32),
        scratch_types=[
            pltpu.VMEM((_CH,), jnp.int32),
            pltpu.VMEM((_CH,), jnp.int32),
            pltpu.VMEM((_CH,), jnp.int32),
            pltpu.VMEM((_CH,), jnp.int32),
            pltpu.VMEM((_CH, 16), jnp.float32),
            pltpu.VMEM((_CH, 16), jnp.float32),
            pltpu.VMEM((_CH, d), jnp.float32),
            pltpu.VMEM((_CH, d), jnp.float32),
            pltpu.VMEM((16, d), jnp.float32),
            pltpu.VMEM_SHARED((npad, d), jnp.float32),
            pltpu.SemaphoreType.DMA,
            pltpu.SemaphoreType.DMA,
            pltpu.SemaphoreType.DMA,
            pltpu.SemaphoreType.DMA,
        ],
    )
    def body(gidx_h, dst_h, normx_h, tab_h, out_h,
             gi0, gi1, ds0, ds1, nx0, nx1, rows0, rows1, zero_v, agg_s,
             semm0, semm1, semr0, semr1):
        cid = lax.axis_index("c")
        sid = lax.axis_index("s")
        wid = sid * _NC + cid
        wbase = wid * epw
        gis = (gi0, gi1)
        dss = (ds0, ds1)
        nxs = (nx0, nx1)
        rows = (rows0, rows1)
        semm = (semm0, semm1)
        semr = (semr0, semr1)

        zvec = jnp.zeros((16,), jnp.float32)
        for i in range(16):
            for j in range(d // 16):
                zero_v[i, pl.ds(j * 16, 16)] = zvec

        def zloop(i, c):
            pltpu.sync_copy(zero_v, agg_s.at[pl.ds(sid * rpt + i * 16, 16)])
            return c
        lax.fori_loop(0, rpt // 16, zloop, 0)
        plsc.subcore_barrier()

        def meta_issue(ci, b):
            pltpu.async_copy(gidx_h.at[wid, ci], gis[b], semm[b])
            pltpu.async_copy(dst_h.at[wid, ci], dss[b], semm[b])
            pltpu.async_copy(normx_h.at[pl.ds(wbase + ci * _CH, _CH)],
                             nxs[b], semm[b])

        def meta_wait(ci, b):
            pltpu.make_async_copy(gidx_h.at[wid, ci], gis[b], semm[b]).wait()
            pltpu.make_async_copy(dst_h.at[wid, ci], dss[b], semm[b]).wait()
            pltpu.make_async_copy(normx_h.at[pl.ds(wbase + ci * _CH, _CH)],
                                  nxs[b], semm[b]).wait()

        def rows_issue(b):
            pltpu.async_copy(tab_h.at[gis[b]], rows[b], semr[b])

        def rows_wait(b):
            pltpu.make_async_copy(tab_h.at[gis[b]], rows[b], semr[b]).wait()

        def half(cj, b):
            @pl.when(cj + 1 < nch)
            def _():
                meta_wait(cj + 1, 1 - b)
                rows_issue(1 - b)
            rows_wait(b)

            def scale(e, c2):
                nv = nxs[b][e]
                for j in range(d // 16):
                    sl = pl.ds(j * 16, 16)
                    rows[b][e, sl] = rows[b][e, sl] * nv
                return c2
            lax.fori_loop(0, _CH, scale, 0)
            pltpu.sync_copy(rows[b], agg_s.at[dss[b]], add=True)

            @pl.when(cj + 2 < nch)
            def _():
                meta_issue(cj + 2, b)

        meta_issue(0, 0)
        meta_wait(0, 0)
        meta_issue(1, 1)
        rows_issue(0)

        def pair(i, c):
            ci = i * 2
            half(ci, 0)
            half(ci + 1, 1)
            return c
        lax.fori_loop(0, nch // 2, pair, 0)
        plsc.subcore_barrier()

        pltpu.sync_copy(agg_s.at[pl.ds(sid * rpt, rpt)],
                        out_h.at[cid, pl.ds(sid * rpt, rpt)])

    return body(gidx3, dst3, normx, table_flat)


def kernel(x, edge_index, rel_type, edge_norm, W0, b0, loop0, gate0,
           W1, b1, loop1, gate1):
    n, d = x.shape
    r = W0.shape[0]
    e = rel_type.shape[0]
    npad = -(-n // _BN) * _BN
    eunit = _NW * _CH * 2
    epad = -(-e // eunit) * eunit

    src = edge_index[0]
    dst = edge_index[1]
    rel = rel_type
    norm = edge_norm[:, 0]
    if epad != e:
        pad = epad - e
        src = jnp.pad(src, (0, pad))
        dst = jnp.pad(dst, (0, pad))
        rel = jnp.pad(rel, (0, pad))
        norm = jnp.pad(norm, (0, pad))
    hpad = jnp.pad(x, ((0, npad - n), (0, 0)))
    normx, gidx = _edge_prep(norm, src, rel, epad, npad)
    epw = epad // _NW
    gidx3 = gidx.reshape(_NW, epw // _CH, _CH)
    dst3 = dst.reshape(_NW, epw // _CH, _CH)

    t0 = _make_table(hpad, W0, gate0.reshape(r, 1, d), npad, d, r)
    p0 = _edge_agg(gidx3, dst3, normx, t0.reshape(r * npad, d), npad, d)
    h1 = _combine(p0, hpad, loop0, b0.reshape(1, d), True, npad, d)

    t1 = _make_table(h1, W1, gate1.reshape(r, 1, d), npad, d, r)
    p1 = _edge_agg(gidx3, dst3, normx, t1.reshape(r * npad, d), npad, d)
    out = _combine(p1, h1, loop1, b1.reshape(1, d), False, npad, d)
    return out[:n]


# speedup vs baseline: 10.8084x; 1.2695x over previous
import functools

import jax
import jax.numpy as jnp
from jax import lax
from jax.experimental import pallas as pl
from jax.experimental.pallas import tpu as pltpu
from jax.experimental.pallas import tpu_sc as plsc

_NC = 2
_NS = 16
_NW = _NC * _NS
_BN = 512
_CH = 48


def _table_body(h_ref, w_ref, g_ref, o_ref):
    hb = h_ref[...]
    w = w_ref[0]
    g = g_ref[0]
    t = jnp.dot(hb, w, preferred_element_type=jnp.float32)
    logit = jnp.sum(hb * g, axis=1, keepdims=True)
    o_ref[...] = t * jax.nn.sigmoid(logit)


def _make_table(hpad, W, gate3d, npad, d, r):
    nb_per_r = npad // _BN
    return pl.pallas_call(
        _table_body,
        grid=(r, nb_per_r),
        in_specs=[
            pl.BlockSpec((_BN, d), lambda ri, nb: (nb, 0)),
            pl.BlockSpec((1, d, d), lambda ri, nb: (ri, 0, 0)),
            pl.BlockSpec((1, 1, d), lambda ri, nb: (ri, 0, 0)),
        ],
        out_specs=pl.BlockSpec(
            (_BN, d), lambda ri, nb: (ri * nb_per_r + nb, 0)),
        out_shape=jax.ShapeDtypeStruct((r * npad, d), jnp.float32),
    )(hpad, W, gate3d)


def _edge_prep_body(src_ref, rel_ref, gidx_ref, *, npad):
    gidx_ref[...] = rel_ref[...] * npad + src_ref[...]


def _edge_prep(src, rel, epad, npad):
    eb = 1024
    return pl.pallas_call(
        functools.partial(_edge_prep_body, npad=npad),
        grid=(epad // eb,),
        in_specs=[
            pl.BlockSpec((eb // 128, 128), lambda i: (i, 0)),
            pl.BlockSpec((eb // 128, 128), lambda i: (i, 0)),
        ],
        out_specs=pl.BlockSpec((eb // 128, 128), lambda i: (i, 0)),
        out_shape=jax.ShapeDtypeStruct((epad // 128, 128), jnp.int32),
    )(src.reshape(epad // 128, 128), rel.reshape(epad // 128, 128))


def _combine_body(p_ref, h_ref, loop_ref, b_ref, o_ref, *, act):
    acc = p_ref[0] + p_ref[1] + b_ref[...] + jnp.dot(
        h_ref[...], loop_ref[...], preferred_element_type=jnp.float32)
    if act:
        acc = jnp.maximum(acc, 0.0)
    o_ref[...] = acc


def _combine(partials, hpad, loopW, b2d, act, npad, d):
    return pl.pallas_call(
        functools.partial(_combine_body, act=act),
        grid=(npad // _BN,),
        in_specs=[
            pl.BlockSpec((2, _BN, d), lambda nb: (0, nb, 0)),
            pl.BlockSpec((_BN, d), lambda nb: (nb, 0)),
            pl.BlockSpec((d, d), lambda nb: (0, 0)),
            pl.BlockSpec((1, d), lambda nb: (0, 0)),
        ],
        out_specs=pl.BlockSpec((_BN, d), lambda nb: (nb, 0)),
        out_shape=jax.ShapeDtypeStruct((npad, d), jnp.float32),
    )(partials, hpad, loopW, b2d)


def _edge_agg(gidx, dst, norm, table_flat, npad, d):
    epad = gidx.shape[0]
    epw = epad // _NW
    nch = epw // _CH
    rpt = npad // _NS
    mesh = plsc.VectorSubcoreMesh(
        core_axis_name="c", subcore_axis_name="s", num_cores=_NC)

    @functools.partial(
        pl.kernel, mesh=mesh,
        out_type=jax.ShapeDtypeStruct((_NC, npad, d), jnp.float32),
        scratch_types=[
            pltpu.VMEM((_CH,), jnp.int32),
            pltpu.VMEM((_CH,), jnp.int32),
            pltpu.VMEM((_CH,), jnp.int32),
            pltpu.VMEM((_CH,), jnp.int32),
            pltpu.VMEM((_CH,), jnp.float32),
            pltpu.VMEM((_CH,), jnp.float32),
            pltpu.VMEM((_CH, d), jnp.float32),
            pltpu.VMEM((_CH, d), jnp.float32),
            pltpu.VMEM((16, d), jnp.float32),
            pltpu.VMEM_SHARED((npad, d), jnp.float32),
            pltpu.SemaphoreType.DMA,
            pltpu.SemaphoreType.DMA,
            pltpu.SemaphoreType.DMA,
            pltpu.SemaphoreType.DMA,
        ],
    )
    def body(gidx_h, dst_h, norm_h, tab_h, out_h,
             gi0, gi1, ds0, ds1, nx0, nx1, rows0, rows1, zero_v, agg_s,
             semm0, semm1, semr0, semr1):
        cid = lax.axis_index("c")
        sid = lax.axis_index("s")
        wid = sid * _NC + cid
        wbase = wid * epw
        gis = (gi0, gi1)
        dss = (ds0, ds1)
        nxs = (nx0, nx1)
        rows = (rows0, rows1)
        semm = (semm0, semm1)
        semr = (semr0, semr1)

        zvec = jnp.zeros((16,), jnp.float32)
        for i in range(16):
            for j in range(d // 16):
                zero_v[i, pl.ds(j * 16, 16)] = zvec

        def zloop(i, c):
            pltpu.sync_copy(zero_v, agg_s.at[pl.ds(sid * rpt + i * 16, 16)])
            return c
        lax.fori_loop(0, rpt // 16, zloop, 0)
        plsc.subcore_barrier()

        def meta_issue(ci, b):
            base = pl.ds(wbase + ci * _CH, _CH)
            pltpu.async_copy(gidx_h.at[base], gis[b], semm[b])
            pltpu.async_copy(dst_h.at[base], dss[b], semm[b])
            pltpu.async_copy(norm_h.at[base], nxs[b], semm[b])

        def meta_wait(ci, b):
            base = pl.ds(wbase + ci * _CH, _CH)
            pltpu.make_async_copy(gidx_h.at[base], gis[b], semm[b]).wait()
            pltpu.make_async_copy(dst_h.at[base], dss[b], semm[b]).wait()
            pltpu.make_async_copy(norm_h.at[base], nxs[b], semm[b]).wait()

        def rows_issue(b):
            pltpu.async_copy(tab_h.at[gis[b]], rows[b], semr[b])

        def rows_wait(b):
            pltpu.make_async_copy(tab_h.at[gis[b]], rows[b], semr[b]).wait()

        def half(cj, b):
            @pl.when(cj + 1 < nch)
            def _():
                meta_wait(cj + 1, 1 - b)
                rows_issue(1 - b)
            rows_wait(b)

            dnums = lax.GatherDimensionNumbers(
                offset_dims=(), collapsed_slice_dims=(0,),
                start_index_map=(0,))

            def scale(g, c2):
                nvec = nxs[b][pl.ds(g * 16, 16)]
                for l in range(16):
                    nv = lax.gather(
                        nvec, jnp.full((16, 1), l, jnp.int32), dnums,
                        slice_sizes=(1,),
                        mode=lax.GatherScatterMode.PROMISE_IN_BOUNDS)
                    e = g * 16 + l
                    for j in range(d // 16):
                        sl = pl.ds(j * 16, 16)
                        rows[b][e, sl] = rows[b][e, sl] * nv
                return c2
            lax.fori_loop(0, _CH // 16, scale, 0)
            pltpu.sync_copy(rows[b], agg_s.at[dss[b]], add=True)

            @pl.when(cj + 2 < nch)
            def _():
                meta_issue(cj + 2, b)

        meta_issue(0, 0)
        meta_wait(0, 0)
        meta_issue(1, 1)
        rows_issue(0)

        def pair(i, c):
            ci = i * 2
            half(ci, 0)
            half(ci + 1, 1)
            return c
        lax.fori_loop(0, nch // 2, pair, 0)
        plsc.subcore_barrier()

        pltpu.sync_copy(agg_s.at[pl.ds(sid * rpt, rpt)],
                        out_h.at[cid, pl.ds(sid * rpt, rpt)])

    return body(gidx, dst, norm, table_flat)


def kernel(x, edge_index, rel_type, edge_norm, W0, b0, loop0, gate0,
           W1, b1, loop1, gate1):
    n, d = x.shape
    r = W0.shape[0]
    e = rel_type.shape[0]
    npad = -(-n // _BN) * _BN
    eunit = _NW * _CH * 2
    epad = -(-e // eunit) * eunit

    src = edge_index[0]
    dst = edge_index[1]
    rel = rel_type
    norm = edge_norm[:, 0]
    if epad != e:
        pad = epad - e
        src = jnp.pad(src, (0, pad))
        dst = jnp.pad(dst, (0, pad))
        rel = jnp.pad(rel, (0, pad))
        norm = jnp.pad(norm, (0, pad))
    hpad = jnp.pad(x, ((0, npad - n), (0, 0)))
    gidx = _edge_prep(src, rel, epad, npad).reshape(epad)

    t0 = _make_table(hpad, W0, gate0.reshape(r, 1, d), npad, d, r)
    p0 = _edge_agg(gidx, dst, norm, t0, npad, d)
    h1 = _combine(p0, hpad, loop0, b0.reshape(1, d), True, npad, d)

    t1 = _make_table(h1, W1, gate1.reshape(r, 1, d), npad, d, r)
    p1 = _edge_agg(gidx, dst, norm, t1, npad, d)
    out = _combine(p1, h1, loop1, b1.reshape(1, d), False, npad, d)
    return out[:n]


# speedup vs baseline: 17.4923x; 1.6184x over previous
import functools

import jax
import jax.numpy as jnp
from jax import lax
from jax.experimental import pallas as pl
from jax.experimental.pallas import tpu as pltpu
from jax.experimental.pallas import tpu_sc as plsc

_NC = 2
_NS = 16
_NW = _NC * _NS
_BN = 512
_CH = 48


def _table_body(h_ref, w_ref, g_ref, o_ref):
    hb = h_ref[...]
    w = w_ref[0]
    g = g_ref[0]
    t = jnp.dot(hb, w, preferred_element_type=jnp.float32)
    logit = jnp.sum(hb * g, axis=1, keepdims=True)
    o_ref[...] = t * jax.nn.sigmoid(logit)


def _make_table(hpad, W, gate3d, npad, d, r):
    bnt = 2048
    nb_per_r = npad // bnt
    return pl.pallas_call(
        _table_body,
        grid=(nb_per_r, r),
        in_specs=[
            pl.BlockSpec((bnt, d), lambda nb, ri: (nb, 0)),
            pl.BlockSpec((1, d, d), lambda nb, ri: (ri, 0, 0)),
            pl.BlockSpec((1, 1, d), lambda nb, ri: (ri, 0, 0)),
        ],
        out_specs=pl.BlockSpec(
            (bnt, d), lambda nb, ri: (ri * nb_per_r + nb, 0)),
        out_shape=jax.ShapeDtypeStruct((r * npad, d), jnp.float32),
    )(hpad, W, gate3d)


def _edge_prep_body(src_ref, rel_ref, gidx_ref, *, npad):
    gidx_ref[...] = rel_ref[...] * npad + src_ref[...]


def _edge_prep(src, rel, epad, npad):
    er = epad // 128
    return pl.pallas_call(
        functools.partial(_edge_prep_body, npad=npad),
        out_shape=jax.ShapeDtypeStruct((er, 128), jnp.int32),
    )(src.reshape(er, 128), rel.reshape(er, 128))


def _combine_body(p_ref, h_ref, loop_ref, b_ref, o_ref, *, act):
    acc = p_ref[0] + p_ref[1] + b_ref[...] + jnp.dot(
        h_ref[...], loop_ref[...], preferred_element_type=jnp.float32)
    if act:
        acc = jnp.maximum(acc, 0.0)
    o_ref[...] = acc


def _combine(partials, hpad, loopW, b2d, act, npad, d):
    return pl.pallas_call(
        functools.partial(_combine_body, act=act),
        grid=(npad // _BN,),
        in_specs=[
            pl.BlockSpec((2, _BN, d), lambda nb: (0, nb, 0)),
            pl.BlockSpec((_BN, d), lambda nb: (nb, 0)),
            pl.BlockSpec((d, d), lambda nb: (0, 0)),
            pl.BlockSpec((1, d), lambda nb: (0, 0)),
        ],
        out_specs=pl.BlockSpec((_BN, d), lambda nb: (nb, 0)),
        out_shape=jax.ShapeDtypeStruct((npad, d), jnp.float32),
    )(partials, hpad, loopW, b2d)


def _edge_agg(gidx, dst, norm, table_flat, npad, d):
    epad = gidx.shape[0]
    epw = epad // _NW
    nch = epw // _CH
    rpt = npad // _NS
    mesh = plsc.VectorSubcoreMesh(
        core_axis_name="c", subcore_axis_name="s", num_cores=_NC)

    @functools.partial(
        pl.kernel, mesh=mesh,
        out_type=jax.ShapeDtypeStruct((_NC, npad, d), jnp.float32),
        scratch_types=[
            pltpu.VMEM((_CH,), jnp.int32),
            pltpu.VMEM((_CH,), jnp.int32),
            pltpu.VMEM((_CH,), jnp.int32),
            pltpu.VMEM((_CH,), jnp.int32),
            pltpu.VMEM((_CH,), jnp.float32),
            pltpu.VMEM((_CH,), jnp.float32),
            pltpu.VMEM((_CH,), jnp.int32),
            pltpu.VMEM((_CH,), jnp.int32),
            pltpu.VMEM((_CH, d), jnp.float32),
            pltpu.VMEM((_CH, d), jnp.float32),
            pltpu.VMEM((16, d), jnp.float32),
            pltpu.VMEM_SHARED((npad, d), jnp.float32),
            pltpu.SemaphoreType.DMA,
            pltpu.SemaphoreType.DMA,
            pltpu.SemaphoreType.DMA,
            pltpu.SemaphoreType.DMA,
            pltpu.SemaphoreType.DMA,
            pltpu.SemaphoreType.DMA,
        ],
    )
    def body(gidx_h, dst_h, norm_h, tab_h, out_h,
             gi0, gi1, ds0, ds1, nx0, nx1, dc0, dc1, rows0, rows1, zero_v,
             agg_s, semm0, semm1, semr0, semr1, sems0, sems1):
        cid = lax.axis_index("c")
        sid = lax.axis_index("s")
        wid = sid * _NC + cid
        wbase = wid * epw
        gis = (gi0, gi1)
        dss = (ds0, ds1)
        nxs = (nx0, nx1)
        rows = (rows0, rows1)
        dcs = (dc0, dc1)
        semm = (semm0, semm1)
        semr = (semr0, semr1)
        sems = (sems0, sems1)

        zvec = jnp.zeros((16,), jnp.float32)
        for i in range(16):
            for j in range(d // 16):
                zero_v[i, pl.ds(j * 16, 16)] = zvec

        def zloop(i, c):
            pltpu.sync_copy(zero_v, agg_s.at[pl.ds(sid * rpt + i * 16, 16)])
            return c
        lax.fori_loop(0, rpt // 16, zloop, 0)
        plsc.subcore_barrier()

        def meta_issue(ci, b):
            base = pl.ds(wbase + ci * _CH, _CH)
            pltpu.async_copy(gidx_h.at[base], gis[b], semm[b])
            pltpu.async_copy(dst_h.at[base], dss[b], semm[b])
            pltpu.async_copy(norm_h.at[base], nxs[b], semm[b])

        def meta_wait(ci, b):
            base = pl.ds(wbase + ci * _CH, _CH)
            pltpu.make_async_copy(gidx_h.at[base], gis[b], semm[b]).wait()
            pltpu.make_async_copy(dst_h.at[base], dss[b], semm[b]).wait()
            pltpu.make_async_copy(norm_h.at[base], nxs[b], semm[b]).wait()

        def rows_issue(b):
            pltpu.async_copy(tab_h.at[gis[b]], rows[b], semr[b])

        def rows_wait(b):
            pltpu.make_async_copy(tab_h.at[gis[b]], rows[b], semr[b]).wait()

        def scat_issue(b):
            for k in range(_CH // 16):
                sl = pl.ds(k * 16, 16)
                dcs[b][sl] = dss[b][sl]
            pltpu.async_copy(rows[b], agg_s.at[dcs[b]], sems[b], add=True)

        def scat_wait(b):
            pltpu.make_async_copy(rows[b], agg_s.at[dcs[b]], sems[b]).wait()

        dnums = lax.GatherDimensionNumbers(
            offset_dims=(), collapsed_slice_dims=(0,),
            start_index_map=(0,))

        def half(cj, b):
            @pl.when(cj >= 1)
            def _():
                scat_wait(1 - b)

            @pl.when(cj + 1 < nch)
            def _():
                meta_wait(cj + 1, 1 - b)
                rows_issue(1 - b)
            rows_wait(b)

            def scale(g, c2):
                nvec = nxs[b][pl.ds(g * 16, 16)]
                for l in range(16):
                    nv = lax.gather(
                        nvec, jnp.full((16, 1), l, jnp.int32), dnums,
                        slice_sizes=(1,),
                        mode=lax.GatherScatterMode.PROMISE_IN_BOUNDS)
                    e = g * 16 + l
                    for j in range(d // 16):
                        sl = pl.ds(j * 16, 16)
                        rows[b][e, sl] = rows[b][e, sl] * nv
                return c2
            lax.fori_loop(0, _CH // 16, scale, 0)
            scat_issue(b)

            @pl.when(cj + 2 < nch)
            def _():
                meta_issue(cj + 2, b)

        meta_issue(0, 0)
        meta_wait(0, 0)
        meta_issue(1, 1)
        rows_issue(0)

        def pair(i, c):
            ci = i * 2
            half(ci, 0)
            half(ci + 1, 1)
            return c
        lax.fori_loop(0, nch // 2, pair, 0)
        scat_wait(1)
        plsc.subcore_barrier()

        pltpu.sync_copy(agg_s.at[pl.ds(sid * rpt, rpt)],
                        out_h.at[cid, pl.ds(sid * rpt, rpt)])

    return body(gidx, dst, norm, table_flat)


def kernel(x, edge_index, rel_type, edge_norm, W0, b0, loop0, gate0,
           W1, b1, loop1, gate1):
    n, d = x.shape
    r = W0.shape[0]
    e = rel_type.shape[0]
    npad = -(-n // _BN) * _BN
    eunit = _NW * _CH * 2
    epad = -(-e // eunit) * eunit

    src = edge_index[0]
    dst = edge_index[1]
    rel = rel_type
    norm = edge_norm[:, 0]
    if epad != e:
        pad = epad - e
        src = jnp.pad(src, (0, pad))
        dst = jnp.pad(dst, (0, pad))
        rel = jnp.pad(rel, (0, pad))
        norm = jnp.pad(norm, (0, pad))
    hpad = jnp.pad(x, ((0, npad - n), (0, 0)))
    gidx = _edge_prep(src, rel, epad, npad).reshape(epad)

    t0 = _make_table(hpad, W0, gate0.reshape(r, 1, d), npad, d, r)
    p0 = _edge_agg(gidx, dst, norm, t0, npad, d)
    h1 = _combine(p0, hpad, loop0, b0.reshape(1, d), True, npad, d)

    t1 = _make_table(h1, W1, gate1.reshape(r, 1, d), npad, d, r)
    p1 = _edge_agg(gidx, dst, norm, t1, npad, d)
    out = _combine(p1, h1, loop1, b1.reshape(1, d), False, npad, d)
    return out[:n]
